# Initial kernel scaffold; baseline (speedup 1.0000x reference)
import functools

import jax
import jax.numpy as jnp
from jax import lax
from jax.experimental import pallas as pl
from jax.experimental.pallas import tpu as pltpu
from jax.experimental.pallas import tpu_sc as plsc

N = 10000
D = 256
H = 128
E = 160000
N_TILES = 16
N_CORES = 2
CHUNK = 128
CHUNKS_PER_TILE = 79
EPAD = N_TILES * CHUNKS_PER_TILE * CHUNK
NROW = 10016
ROWS_PER_TILE = NROW // N_TILES
DEG_W = 8


def _sc_body(x_il, sidx, didx, zeros_acc, zeros_deg, ones_blk,
             agg_out, deg_out,
             idxs_v, idxd_v, rows_v, ones_v, acc_sh, deg_sh, sem):
    c = lax.axis_index("c")
    s = lax.axis_index("s")
    base = s * ROWS_PER_TILE
    pltpu.sync_copy(zeros_acc.at[pl.ds(base, ROWS_PER_TILE)],
                    acc_sh.at[pl.ds(base, ROWS_PER_TILE)])

    @pl.when(c == 0)
    def _():
        pltpu.sync_copy(zeros_deg.at[pl.ds(base, ROWS_PER_TILE)],
                        deg_sh.at[pl.ds(base, ROWS_PER_TILE)])
        pltpu.sync_copy(ones_blk, ones_v)

    pltpu.sync_copy(sidx.at[c, s], idxs_v)
    pltpu.sync_copy(didx.at[s], idxd_v)
    plsc.subcore_barrier()

    def chunk(j, carry):
        pltpu.async_copy(x_il.at[idxs_v.at[j]], rows_v, sem).wait()
        pltpu.sync_copy(rows_v, acc_sh.at[idxd_v.at[j]], add=True)

        @pl.when(c == 0)
        def _():
            pltpu.sync_copy(ones_v, deg_sh.at[idxd_v.at[j]], add=True)

        return carry

    lax.fori_loop(0, CHUNKS_PER_TILE, chunk, 0)
    plsc.subcore_barrier()

    pltpu.sync_copy(acc_sh.at[pl.ds(base, ROWS_PER_TILE)],
                    agg_out.at[c, pl.ds(base, ROWS_PER_TILE)])

    @pl.when(c == 0)
    def _():
        pltpu.sync_copy(deg_sh.at[pl.ds(base, ROWS_PER_TILE)],
                        deg_out.at[pl.ds(base, ROWS_PER_TILE)])


_sc_agg_call = pl.kernel(
    _sc_body,
    out_type=(
        jax.ShapeDtypeStruct((N_CORES, NROW, H), jnp.float32),
        jax.ShapeDtypeStruct((NROW, DEG_W), jnp.float32),
    ),
    mesh=plsc.VectorSubcoreMesh(core_axis_name="c", subcore_axis_name="s"),
    scratch_types=[
        pltpu.VMEM((CHUNKS_PER_TILE, CHUNK), jnp.int32),
        pltpu.VMEM((CHUNKS_PER_TILE, CHUNK), jnp.int32),
        pltpu.VMEM((CHUNK, H), jnp.float32),
        pltpu.VMEM((CHUNK, DEG_W), jnp.float32),
        pltpu.VMEM_SHARED((NROW, H), jnp.float32),
        pltpu.VMEM_SHARED((NROW, DEG_W), jnp.float32),
        pltpu.SemaphoreType.DMA,
    ],
)


def _sc_agg(x_src, edge_index):
    src = edge_index[0].astype(jnp.int32)
    dst = edge_index[1].astype(jnp.int32)
    src_p = jnp.pad(src, (0, EPAD - E))
    dst_p = jnp.pad(dst, (0, EPAD - E), constant_values=N)
    sidx = jnp.stack([2 * src_p, 2 * src_p + 1]).reshape(
        N_CORES, N_TILES, CHUNKS_PER_TILE, CHUNK)
    didx = dst_p.reshape(N_TILES, CHUNKS_PER_TILE, CHUNK)
    x_il = x_src.reshape(2 * N, H)
    zeros_acc = jnp.zeros((NROW, H), jnp.float32)
    zeros_deg = jnp.zeros((NROW, DEG_W), jnp.float32)
    ones_blk = jnp.ones((CHUNK, DEG_W), jnp.float32)
    return _sc_agg_call(x_il, sidx, didx, zeros_acc, zeros_deg, ones_blk)


TC_BLK = 1000


def _tc_body(a0_ref, a1_ref, deg_ref, x_ref, wl_ref, wr_ref, bl_ref, o_ref):
    d = jnp.maximum(deg_ref[:, :1], 1.0)
    agg = jnp.concatenate([a0_ref[...], a1_ref[...]], axis=1) / d
    o_ref[...] = (
        jnp.dot(agg, wl_ref[...], preferred_element_type=jnp.float32)
        + jnp.dot(x_ref[...], wr_ref[...], preferred_element_type=jnp.float32)
        + bl_ref[...])


_tc_call = pl.pallas_call(
    _tc_body,
    grid=(N // TC_BLK,),
    in_specs=[
        pl.BlockSpec((TC_BLK, H), lambda i: (i, 0)),
        pl.BlockSpec((TC_BLK, H), lambda i: (i, 0)),
        pl.BlockSpec((TC_BLK, DEG_W), lambda i: (i, 0)),
        pl.BlockSpec((TC_BLK, D), lambda i: (i, 0)),
        pl.BlockSpec((D, D), lambda i: (0, 0)),
        pl.BlockSpec((D, D), lambda i: (0, 0)),
        pl.BlockSpec((1, D), lambda i: (0, 0)),
    ],
    out_specs=pl.BlockSpec((TC_BLK, D), lambda i: (i, 0)),
    out_shape=jax.ShapeDtypeStruct((N, D), jnp.float32),
)


def _tc(agg, deg, x_dst, Wl, Wr, bl):
    return _tc_call(agg[0, :N], agg[1, :N], deg[:N], x_dst,
                    Wl, Wr, bl.reshape(1, D))


def kernel(x_sites, x_wells, edge_index_s2w, edge_index_w2s,
           Wl_s2w, bl_s2w, Wr_s2w, Wl_w2s, bl_w2s, Wr_w2s):
    agg_w, deg_w = _sc_agg(x_sites, edge_index_s2w)
    agg_s, deg_s = _sc_agg(x_wells, edge_index_w2s)
    z_wells = _tc(agg_w, deg_w, x_wells, Wl_s2w, Wr_s2w, bl_s2w)
    z_sites = _tc(agg_s, deg_s, x_sites, Wl_w2s, Wr_w2s, bl_w2s)
    return (z_sites, z_wells)


# speedup vs baseline: 2.8512x; 2.8512x over previous
import jax
import jax.numpy as jnp
from jax import lax
from jax.experimental import pallas as pl
from jax.experimental.pallas import tpu as pltpu
from jax.experimental.pallas import tpu_sc as plsc

N = 10000
D = 256
Q = 64
NQ = 4
E = 160000
N_TILES = 16
N_CORES = 2
CHUNK = 128
CHUNKS_PER_TILE = 79
EPAD = N_TILES * CHUNKS_PER_TILE * CHUNK
NROW = 10112
ROWS_PER_TILE = NROW // N_TILES
DEG_W = 8


def _sc_body(x_q, sidx, didx, zeros_acc, zeros_deg, ones_blk,
             agg_out, deg_out,
             idxs_v, idxd_v, rows_v, ones_v, acc_sh, deg_sh, sem):
    c = lax.axis_index("c")
    s = lax.axis_index("s")
    base = s * ROWS_PER_TILE
    row_sl = pl.ds(base, ROWS_PER_TILE)

    @pl.when(c == 0)
    def _():
        pltpu.sync_copy(ones_blk, ones_v)

    pltpu.sync_copy(didx.at[s], idxd_v)

    for p in range(2):
        q = 2 * p + c
        pltpu.sync_copy(zeros_acc.at[row_sl], acc_sh.at[row_sl])

        if p == 0:
            @pl.when(c == 0)
            def _():
                pltpu.sync_copy(zeros_deg.at[row_sl], deg_sh.at[row_sl])

        pltpu.sync_copy(sidx.at[q, s], idxs_v)
        plsc.subcore_barrier()

        def chunk(j, carry):
            pltpu.async_copy(x_q.at[idxs_v.at[j]], rows_v, sem).wait()
            pltpu.sync_copy(rows_v, acc_sh.at[idxd_v.at[j]], add=True)

            if p == 0:
                @pl.when(c == 0)
                def _():
                    pltpu.sync_copy(ones_v, deg_sh.at[idxd_v.at[j]], add=True)

            return carry

        lax.fori_loop(0, CHUNKS_PER_TILE, chunk, 0)
        plsc.subcore_barrier()

        pltpu.sync_copy(acc_sh.at[row_sl], agg_out.at[q, row_sl])

        if p == 0:
            @pl.when(c == 0)
            def _():
                pltpu.sync_copy(deg_sh.at[row_sl], deg_out.at[row_sl])


_sc_agg_call = pl.kernel(
    _sc_body,
    out_type=(
        jax.ShapeDtypeStruct((NQ, NROW, Q), jnp.float32),
        jax.ShapeDtypeStruct((NROW, DEG_W), jnp.float32),
    ),
    mesh=plsc.VectorSubcoreMesh(core_axis_name="c", subcore_axis_name="s"),
    compiler_params=pltpu.CompilerParams(use_tc_tiling_on_sc=False),
    scratch_types=[
        pltpu.VMEM((CHUNKS_PER_TILE, CHUNK), jnp.int32),
        pltpu.VMEM((CHUNKS_PER_TILE, CHUNK), jnp.int32),
        pltpu.VMEM((CHUNK, Q), jnp.float32),
        pltpu.VMEM((CHUNK, DEG_W), jnp.float32),
        pltpu.VMEM_SHARED((NROW, Q), jnp.float32),
        pltpu.VMEM_SHARED((NROW, DEG_W), jnp.float32),
        pltpu.SemaphoreType.DMA,
    ],
)


def _sc_agg(x_src, edge_index):
    src = edge_index[0].astype(jnp.int32)
    dst = edge_index[1].astype(jnp.int32)
    src_p = jnp.pad(src, (0, EPAD - E))
    dst_p = jnp.pad(dst, (0, EPAD - E), constant_values=N)
    sidx = jnp.stack([4 * src_p + q for q in range(NQ)]).reshape(
        NQ, N_TILES, CHUNKS_PER_TILE, CHUNK)
    didx = dst_p.reshape(N_TILES, CHUNKS_PER_TILE, CHUNK)
    x_q = x_src.reshape(NQ * N, Q)
    zeros_acc = jnp.zeros((NROW, Q), jnp.float32)
    zeros_deg = jnp.zeros((NROW, DEG_W), jnp.float32)
    ones_blk = jnp.ones((CHUNK, DEG_W), jnp.float32)
    return _sc_agg_call(x_q, sidx, didx, zeros_acc, zeros_deg, ones_blk)


TC_BLK = 1000


def _tc_body(a0_ref, a1_ref, a2_ref, a3_ref, deg_ref, x_ref,
             wl_ref, wr_ref, bl_ref, o_ref):
    d = jnp.maximum(deg_ref[:, :1], 1.0)
    agg = jnp.concatenate(
        [a0_ref[...], a1_ref[...], a2_ref[...], a3_ref[...]], axis=1) / d
    o_ref[...] = (
        jnp.dot(agg, wl_ref[...], preferred_element_type=jnp.float32)
        + jnp.dot(x_ref[...], wr_ref[...], preferred_element_type=jnp.float32)
        + bl_ref[...])


_tc_call = pl.pallas_call(
    _tc_body,
    grid=(N // TC_BLK,),
    in_specs=[
        pl.BlockSpec((TC_BLK, Q), lambda i: (i, 0)),
        pl.BlockSpec((TC_BLK, Q), lambda i: (i, 0)),
        pl.BlockSpec((TC_BLK, Q), lambda i: (i, 0)),
        pl.BlockSpec((TC_BLK, Q), lambda i: (i, 0)),
        pl.BlockSpec((TC_BLK, DEG_W), lambda i: (i, 0)),
        pl.BlockSpec((TC_BLK, D), lambda i: (i, 0)),
        pl.BlockSpec((D, D), lambda i: (0, 0)),
        pl.BlockSpec((D, D), lambda i: (0, 0)),
        pl.BlockSpec((1, D), lambda i: (0, 0)),
    ],
    out_specs=pl.BlockSpec((TC_BLK, D), lambda i: (i, 0)),
    out_shape=jax.ShapeDtypeStruct((N, D), jnp.float32),
)


def _tc(agg, deg, x_dst, Wl, Wr, bl):
    return _tc_call(agg[0, :N], agg[1, :N], agg[2, :N], agg[3, :N],
                    deg[:N], x_dst, Wl, Wr, bl.reshape(1, D))


def kernel(x_sites, x_wells, edge_index_s2w, edge_index_w2s,
           Wl_s2w, bl_s2w, Wr_s2w, Wl_w2s, bl_w2s, Wr_w2s):
    agg_w, deg_w = _sc_agg(x_sites, edge_index_s2w)
    agg_s, deg_s = _sc_agg(x_wells, edge_index_w2s)
    z_wells = _tc(agg_w, deg_w, x_wells, Wl_s2w, Wr_s2w, bl_s2w)
    z_sites = _tc(agg_s, deg_s, x_sites, Wl_w2s, Wr_w2s, bl_w2s)
    return (z_sites, z_wells)
